# baseline (speedup 1.0000x reference)
import functools

import jax
import jax.numpy as jnp
from jax import lax
from jax.experimental import pallas as pl
from jax.experimental.pallas import tpu as pltpu
from jax.experimental.pallas import tpu_sc as plsc

_VOCAB = 10000
_HIDDEN = 1024
_BATCH = 16384

_NC, _NS, _L = 2, 16, 16
_NW = _NC * _NS
_ROWS_PER_W = _BATCH // _NW
_CHUNK = 32
_NCHUNK = _ROWS_PER_W // _CHUNK

_BM = 1000


def _mm_body(x_ref, w_ref, b_ref, o_ref):
    o_ref[...] = lax.dot_general(
        x_ref[...], w_ref[...],
        dimension_numbers=(((1,), (1,)), ((), ())),
        preferred_element_type=jnp.float32,
    ) + b_ref[...]


def _transform_table(emb, W, b):
    return pl.pallas_call(
        _mm_body,
        grid=(_VOCAB // _BM,),
        in_specs=[
            pl.BlockSpec((_BM, _HIDDEN), lambda i: (i, 0)),
            pl.BlockSpec((_HIDDEN, _HIDDEN), lambda i: (0, 0)),
            pl.BlockSpec((1, _HIDDEN), lambda i: (0, 0)),
        ],
        out_specs=pl.BlockSpec((_BM, _HIDDEN), lambda i: (i, 0)),
        out_shape=jax.ShapeDtypeStruct((_VOCAB, _HIDDEN), jnp.float32),
    )(emb, W, b.reshape(1, _HIDDEN))


@functools.cache
def _make_sc_gather_add():
    @functools.partial(
        pl.kernel,
        out_type=jax.ShapeDtypeStruct((_BATCH, _HIDDEN), jnp.float32),
        mesh=plsc.VectorSubcoreMesh(core_axis_name="c", subcore_axis_name="s"),
        scratch_types=[
            pltpu.VMEM((_CHUNK,), jnp.int32),
            pltpu.VMEM((_CHUNK, _HIDDEN), jnp.float32),
            pltpu.VMEM((_CHUNK, _HIDDEN), jnp.float32),
            pltpu.SemaphoreType.DMA,
        ],
    )
    def _sc_gather_add(table_hbm, ids_hbm, noise_hbm, out_hbm,
                       idx_v, rows_v, noise_v, sem):
        wid = lax.axis_index("s") * _NC + lax.axis_index("c")
        base = wid * _ROWS_PER_W

        def chunk(c, _):
            gbase = base + c * _CHUNK
            pltpu.sync_copy(ids_hbm.at[pl.ds(gbase, _CHUNK)], idx_v)
            pltpu.async_copy(table_hbm.at[idx_v], rows_v, sem).wait()
            pltpu.sync_copy(noise_hbm.at[pl.ds(gbase, _CHUNK)], noise_v)

            def row(r, _):
                def col(j, _):
                    sl = pl.ds(j * _L, _L)
                    rows_v[r, sl] = rows_v[r, sl] + noise_v[r, sl] * 0.1
                    return 0
                return lax.fori_loop(0, _HIDDEN // _L, col, 0)

            lax.fori_loop(0, _CHUNK, row, 0)
            pltpu.sync_copy(rows_v, out_hbm.at[pl.ds(gbase, _CHUNK)])
            return 0

        lax.fori_loop(0, _NCHUNK, chunk, 0)

    return _sc_gather_add


def kernel(input_ids, emb_table, W, b):
    ids = input_ids[:, -1]
    noise = jax.random.normal(jax.random.key(1), (_BATCH, _HIDDEN),
                              dtype=jnp.float32)
    table = _transform_table(emb_table, W, b)
    return _make_sc_gather_add()(table, ids, noise)


# speedup vs baseline: 1.2975x; 1.2975x over previous
import functools

import jax
import jax.numpy as jnp
from jax import lax
from jax.experimental import pallas as pl
from jax.experimental.pallas import tpu as pltpu
from jax.experimental.pallas import tpu_sc as plsc

_VOCAB = 10000
_HIDDEN = 1024
_BATCH = 16384

_NC, _NS, _L = 2, 16, 16
_NW = _NC * _NS
_ROWS_PER_W = _BATCH // _NW
_CHUNK = 16
_NBUF = 2
_NCHUNK = _ROWS_PER_W // _CHUNK

_BM = 1000


def _mm_body(x_ref, w_ref, b_ref, o_ref):
    o_ref[...] = lax.dot_general(
        x_ref[...].astype(jnp.bfloat16), w_ref[...].astype(jnp.bfloat16),
        dimension_numbers=(((1,), (1,)), ((), ())),
        preferred_element_type=jnp.float32,
    ) + b_ref[...]


def _transform_table(emb, W, b):
    return pl.pallas_call(
        _mm_body,
        grid=(_VOCAB // _BM,),
        in_specs=[
            pl.BlockSpec((_BM, _HIDDEN), lambda i: (i, 0)),
            pl.BlockSpec((_HIDDEN, _HIDDEN), lambda i: (0, 0)),
            pl.BlockSpec((1, _HIDDEN), lambda i: (0, 0)),
        ],
        out_specs=pl.BlockSpec((_BM, _HIDDEN), lambda i: (i, 0)),
        out_shape=jax.ShapeDtypeStruct((_VOCAB, _HIDDEN), jnp.float32),
    )(emb, W, b.reshape(1, _HIDDEN))


@functools.cache
def _make_sc_gather_add():
    @functools.partial(
        pl.kernel,
        out_type=jax.ShapeDtypeStruct((_BATCH, _HIDDEN), jnp.float32),
        mesh=plsc.VectorSubcoreMesh(core_axis_name="c", subcore_axis_name="s"),
        scratch_types=[
            pltpu.VMEM((_ROWS_PER_W,), jnp.int32),
            pltpu.VMEM((_NBUF, _CHUNK, _HIDDEN), jnp.float32),
            pltpu.VMEM((_NBUF, _CHUNK, _HIDDEN), jnp.float32),
            pltpu.VMEM((_NBUF, _CHUNK, _HIDDEN), jnp.float32),
            pltpu.SemaphoreType.DMA,
            pltpu.SemaphoreType.DMA,
            pltpu.SemaphoreType.DMA,
            pltpu.SemaphoreType.DMA,
            pltpu.SemaphoreType.DMA,
            pltpu.SemaphoreType.DMA,
        ],
    )
    def _sc_gather_add(table_hbm, ids_hbm, noise_hbm, out_hbm,
                       idx_v, rows_v, noise_v, out_v,
                       gsem0, gsem1, nsem0, nsem1, wsem0, wsem1):
        gsem = (gsem0, gsem1)
        nsem = (nsem0, nsem1)
        wsem = (wsem0, wsem1)
        wid = lax.axis_index("s") * _NC + lax.axis_index("c")
        base = wid * _ROWS_PER_W
        pltpu.sync_copy(ids_hbm.at[pl.ds(base, _ROWS_PER_W)], idx_v)

        def fetch(c, b):
            idxs = idx_v[pl.ds(c * _CHUNK, _CHUNK)]
            pltpu.async_copy(table_hbm.at[idxs], rows_v.at[b], gsem[b])
            pltpu.async_copy(noise_hbm.at[pl.ds(base + c * _CHUNK, _CHUNK)],
                             noise_v.at[b], nsem[b])

        for b in range(_NBUF):
            fetch(b, b)

        def outer(i, _):
            cc = i * _NBUF
            for b in range(_NBUF):
                c = cc + b
                pltpu.make_async_copy(
                    table_hbm.at[idx_v[pl.ds(0, _CHUNK)]],
                    rows_v.at[b], gsem[b]).wait()
                pltpu.make_async_copy(
                    noise_hbm.at[pl.ds(0, _CHUNK)],
                    noise_v.at[b], nsem[b]).wait()

                @pl.when(cc >= _NBUF)
                def _(b=b):
                    pltpu.make_async_copy(
                        out_v.at[b], out_hbm.at[pl.ds(0, _CHUNK)],
                        wsem[b]).wait()

                def row(r, _, b=b):
                    for j in range(_HIDDEN // _L):
                        sl = pl.ds(j * _L, _L)
                        out_v[b, r, sl] = rows_v[b, r, sl] + noise_v[b, r, sl] * 0.1
                    return 0

                lax.fori_loop(0, _CHUNK, row, 0, unroll=2)
                pltpu.async_copy(
                    out_v.at[b], out_hbm.at[pl.ds(base + c * _CHUNK, _CHUNK)],
                    wsem[b])

                @pl.when(cc + _NBUF < _NCHUNK)
                def _(c=c, b=b):
                    fetch(c + _NBUF, b)
            return 0

        lax.fori_loop(0, _NCHUNK // _NBUF, outer, 0)
        for b in range(_NBUF):
            pltpu.make_async_copy(out_v.at[b], out_hbm.at[pl.ds(0, _CHUNK)],
                                  wsem[b]).wait()

    return _sc_gather_add


def kernel(input_ids, emb_table, W, b):
    ids = input_ids[:, -1]
    noise = jax.random.normal(jax.random.key(1), (_BATCH, _HIDDEN),
                              dtype=jnp.float32)
    table = _transform_table(emb_table, W, b)
    return _make_sc_gather_add()(table, ids, noise)
